# baseline (speedup 1.0000x reference)
import functools

import jax
import jax.numpy as jnp
from jax.experimental import pallas as pl
from jax.experimental.pallas import tpu as pltpu
from jax.experimental.pallas import tpu_sc as plsc


def _fps_body(npoint, Nr, B, xyz_ref, inds_ref, nxyz_ref, dist_ref):
    r_io = jax.lax.broadcasted_iota(jnp.int32, (Nr, 128), 0)
    for b in range(B):
        dist_ref[b] = jnp.full((Nr, 128), 1e10, jnp.float32)
    l_io = jax.lax.broadcasted_iota(jnp.int32, (1, 128), 1)
    BIGI = jnp.int32(Nr * 128)

    def step(t, fars):
        tr = t // 128
        tc = t - tr * 128
        hitc = l_io == tc
        new_fars = []
        for b in range(B):
            far = fars[b]
            r = far // 128
            c = far - r * 128
            csel = l_io == c
            cx = jnp.sum(jnp.where(csel, xyz_ref[b, 0, pl.ds(r, 1), :], 0.0))
            cy = jnp.sum(jnp.where(csel, xyz_ref[b, 1, pl.ds(r, 1), :], 0.0))
            cz = jnp.sum(jnp.where(csel, xyz_ref[b, 2, pl.ds(r, 1), :], 0.0))
            inds_ref[b, pl.ds(tr, 1), :] = jnp.where(
                hitc, far, inds_ref[b, pl.ds(tr, 1), :])
            nxyz_ref[b, 0, pl.ds(tr, 1), :] = jnp.where(
                hitc, cx, nxyz_ref[b, 0, pl.ds(tr, 1), :])
            nxyz_ref[b, 1, pl.ds(tr, 1), :] = jnp.where(
                hitc, cy, nxyz_ref[b, 1, pl.ds(tr, 1), :])
            nxyz_ref[b, 2, pl.ds(tr, 1), :] = jnp.where(
                hitc, cz, nxyz_ref[b, 2, pl.ds(tr, 1), :])
            dx = xyz_ref[b, 0] - cx
            dy = xyz_ref[b, 1] - cy
            dz = xyz_ref[b, 2] - cz
            d = dx * dx + dy * dy + dz * dz
            nd = jnp.minimum(dist_ref[b], d)
            dist_ref[b] = nd
            rowmax = jnp.max(nd, axis=0, keepdims=True)
            rowarg = jnp.min(jnp.where(nd == rowmax, r_io, jnp.int32(Nr)),
                             axis=0, keepdims=True)
            m = jnp.max(rowmax)
            cand = jnp.where(rowmax == m, rowarg * 128 + l_io, BIGI)
            new_fars.append(jnp.min(cand))
        return tuple(new_fars)

    jax.lax.fori_loop(0, npoint, step, (jnp.int32(0),) * B)


def _fps(xyz, npoint):
    B, N, _ = xyz.shape
    Nr = max(8, -(-N // 128))
    Np = Nr * 128
    rows = npoint // 128
    if Np > N:
        pad = jnp.broadcast_to(xyz[:, 0:1, :], (B, Np - N, 3))
        xyz = jnp.concatenate([xyz, pad], axis=1)
    xyz_p = jnp.transpose(xyz, (0, 2, 1)).reshape(B, 3, Nr, 128)
    inds, nxyz = pl.pallas_call(
        functools.partial(_fps_body, npoint, Nr, B),
        grid=(1,),
        in_specs=[pl.BlockSpec((B, 3, Nr, 128), lambda i: (0, 0, 0, 0))],
        out_specs=[pl.BlockSpec((B, rows, 128), lambda i: (0, 0, 0)),
                   pl.BlockSpec((B, 3, rows, 128), lambda i: (0, 0, 0, 0))],
        out_shape=[jax.ShapeDtypeStruct((B, rows, 128), jnp.int32),
                   jax.ShapeDtypeStruct((B, 3, rows, 128), jnp.float32)],
        scratch_shapes=[pltpu.VMEM((B, Nr, 128), jnp.float32)],
    )(xyz_p)
    return inds.reshape(B, npoint), nxyz.reshape(B, 3, npoint)


def _mlp_body(K, inv_r, g_ref, q_ref, w1_ref, b1_ref, w2_ref, b2_ref,
              w3_ref, b3_ref, out_ref):
    SB = g_ref.shape[1]
    D = g_ref.shape[3]
    g = g_ref[0]
    q = q_ref[0][:, None, :]
    col = jax.lax.broadcasted_iota(jnp.int32, (SB, K, D), 2)
    h0 = jnp.where(col < 3, (g - q) * inv_r, g)
    xx = h0.reshape(SB * K, D)
    h = jnp.maximum(jnp.dot(xx, w1_ref[...],
                            preferred_element_type=jnp.float32) + b1_ref[...], 0.0)
    h = jnp.maximum(jnp.dot(h, w2_ref[...],
                            preferred_element_type=jnp.float32) + b2_ref[...], 0.0)
    h = jnp.maximum(jnp.dot(h, w3_ref[...],
                            preferred_element_type=jnp.float32) + b3_ref[...], 0.0)
    C3 = h.shape[-1]
    out_ref[0] = jnp.max(h.reshape(SB, K, C3), axis=1)


def _mlp_max(g, q, params, D, radius):
    B, S, K, _ = g.shape
    SB = min(S, 128)
    (w1, b1), (w2, b2), (w3, b3) = params
    cin = w1.shape[0]
    w1p = jnp.concatenate([w1, jnp.zeros((D - cin, w1.shape[1]), jnp.float32)], axis=0)
    C3 = w3.shape[1]
    out = pl.pallas_call(
        functools.partial(_mlp_body, K, 1.0 / radius),
        grid=(B, S // SB),
        in_specs=[
            pl.BlockSpec((1, SB, K, D), lambda b, s: (b, s, 0, 0)),
            pl.BlockSpec((1, SB, D), lambda b, s: (b, s, 0)),
            pl.BlockSpec(w1p.shape, lambda b, s: (0, 0)),
            pl.BlockSpec((1, b1.shape[0]), lambda b, s: (0, 0)),
            pl.BlockSpec(w2.shape, lambda b, s: (0, 0)),
            pl.BlockSpec((1, b2.shape[0]), lambda b, s: (0, 0)),
            pl.BlockSpec(w3.shape, lambda b, s: (0, 0)),
            pl.BlockSpec((1, b3.shape[0]), lambda b, s: (0, 0)),
        ],
        out_specs=pl.BlockSpec((1, SB, C3), lambda b, s: (b, s, 0)),
        out_shape=jax.ShapeDtypeStruct((B, S, C3), jnp.float32),
    )(g, q, w1p, b1[None, :], w2, b2[None, :], w3, b3[None, :])
    return out


def _fp_body(S2, f1_ref, x1_ref, x2t_ref, f2_ref, w1_ref, b1_ref,
             w2_ref, b2_ref, out_ref):
    x1 = x1_ref[0]
    x2 = x2t_ref[0]
    S1 = x1.shape[0]
    d = None
    for c in range(3):
        t = x1[:, c:c + 1] - x2[c:c + 1, :]
        t = t * t
        d = t if d is None else d + t
    colj = jax.lax.broadcasted_iota(jnp.int32, (S1, S2), 1)
    f2 = f2_ref[0]
    dd = d
    invs = []
    rows = []
    for _ in range(3):
        mn = jnp.min(dd, axis=1, keepdims=True)
        im = jnp.min(jnp.where(dd == mn, colj, S2), axis=1, keepdims=True)
        dd = jnp.where(colj == im, 1e30, dd)
        invs.append(1.0 / jnp.maximum(mn, 1e-10))
        oh = (colj == im).astype(jnp.float32)
        rows.append(jnp.dot(oh, f2, preferred_element_type=jnp.float32))
    wsum = (invs[0] + invs[1]) + invs[2]
    interp = ((rows[0] * (invs[0] / wsum) + rows[1] * (invs[1] / wsum))
              + rows[2] * (invs[2] / wsum))
    h = jnp.concatenate([interp, f1_ref[0]], axis=-1)
    h = jnp.maximum(jnp.dot(h, w1_ref[...],
                            preferred_element_type=jnp.float32) + b1_ref[...], 0.0)
    h = jnp.maximum(jnp.dot(h, w2_ref[...],
                            preferred_element_type=jnp.float32) + b2_ref[...], 0.0)
    out_ref[0] = h


def _fp(x1, f1, x2t, f2, params):
    B, S1, C1 = f1.shape
    S2 = f2.shape[1]
    (w1, b1), (w2, b2) = params
    Cout = w2.shape[1]
    out = pl.pallas_call(
        functools.partial(_fp_body, S2),
        grid=(B,),
        in_specs=[
            pl.BlockSpec((1, S1, C1), lambda b: (b, 0, 0)),
            pl.BlockSpec((1, S1, 3), lambda b: (b, 0, 0)),
            pl.BlockSpec((1, 3, S2), lambda b: (b, 0, 0)),
            pl.BlockSpec((1, S2, f2.shape[2]), lambda b: (b, 0, 0)),
            pl.BlockSpec(w1.shape, lambda b: (0, 0)),
            pl.BlockSpec((1, b1.shape[0]), lambda b: (0, 0)),
            pl.BlockSpec(w2.shape, lambda b: (0, 0)),
            pl.BlockSpec((1, b2.shape[0]), lambda b: (0, 0)),
        ],
        out_specs=pl.BlockSpec((1, S1, Cout), lambda b: (b, 0, 0)),
        out_shape=jax.ShapeDtypeStruct((B, S1, Cout), jnp.float32),
    )(f1, x1, x2t, f2, w1, b1[None, :], w2, b2[None, :])
    return out


def _ball_group(radius, K, xyz, new_xyz, table):
    B, S, _ = new_xyz.shape
    N = xyz.shape[1]
    D = table.shape[-1]
    NW = 32
    rows_per = (B * S) // NW
    nc = N // 16
    r2 = radius * radius
    mesh = plsc.VectorSubcoreMesh(core_axis_name="c", subcore_axis_name="s")

    @functools.partial(
        pl.kernel, mesh=mesh,
        compiler_params=pltpu.CompilerParams(needs_layout_passes=False),
        out_type=jax.ShapeDtypeStruct((B * S * K, D), jnp.float32),
        scratch_types=[
            pltpu.VMEM((N,), jnp.float32),
            pltpu.VMEM((N,), jnp.float32),
            pltpu.VMEM((N,), jnp.float32),
            pltpu.VMEM((rows_per * 16,), jnp.float32),
            pltpu.VMEM((rows_per * 16,), jnp.float32),
            pltpu.VMEM((rows_per * 16,), jnp.float32),
            pltpu.VMEM((K,), jnp.int32),
            pltpu.VMEM((K,), jnp.int32),
            pltpu.VMEM((K, D), jnp.float32),
            pltpu.VMEM((16,), jnp.int32),
            pltpu.SemaphoreType.DMA,
        ],
    )
    def bq(qx_h, qy_h, qz_h, px_h, py_h, pz_h, tab_h, g_h,
           xv, yv, zv, qxv, qyv, qzv, grp_v, gidx_v, rows_v, dbg_v, sem):
        wid = jax.lax.axis_index("s") * 2 + jax.lax.axis_index("c")
        base = wid * rows_per
        b = base // S
        pltpu.sync_copy(px_h.at[pl.ds(b * N, N)], xv)
        pltpu.sync_copy(py_h.at[pl.ds(b * N, N)], yv)
        pltpu.sync_copy(pz_h.at[pl.ds(b * N, N)], zv)
        pltpu.sync_copy(qx_h.at[pl.ds(base * 16, rows_per * 16)], qxv)
        pltpu.sync_copy(qy_h.at[pl.ds(base * 16, rows_per * 16)], qyv)
        pltpu.sync_copy(qz_h.at[pl.ds(base * 16, rows_per * 16)], qzv)
        iot = jax.lax.iota(jnp.int32, 16)
        zeros16 = jnp.zeros((16,), jnp.int32)
        bN = b * N

        def row_body(s_local, _):
            qoff = s_local * 16
            qxs = qxv[pl.ds(qoff, 16)]
            qys = qyv[pl.ds(qoff, 16)]
            qzs = qzv[pl.ds(qoff, 16)]

            def chunk(i, cnt_vec):
                off = i * 16
                dx = xv[pl.ds(off, 16)] - qxs
                dy = yv[pl.ds(off, 16)] - qys
                dz = zv[pl.ds(off, 16)] - qzs
                dd = dx * dx + dy * dy + dz * dz
                m = dd <= r2

                def hit(cv):
                    c = plsc.cumsum(jnp.where(m, jnp.int32(1), jnp.int32(0)))
                    pos = cv + c - 1
                    keep = m & (pos < K)
                    pos_st = jnp.where(keep, pos, jnp.int32(0))
                    plsc.store_scatter(grp_v, [pos_st], iot + off, mask=keep)
                    return cv + plsc.all_reduce_population_count(keep)

                return jax.lax.cond(jnp.any(m), hit, lambda cv: cv, cnt_vec)

            cnt = jax.lax.fori_loop(0, nc, chunk, zeros16)
            v0vec = grp_v[pl.ds(0, 16)]
            v0 = plsc.cummax(jnp.where(iot == 0, v0vec, jnp.int32(-1)))
            for kb in range(K // 16):
                sl = pl.ds(kb * 16, 16)
                sel = (iot + kb * 16) >= cnt
                gidx_v[sl] = jnp.where(sel, v0, grp_v[sl]) + bN
            pltpu.async_copy(tab_h.at[gidx_v], rows_v, sem).wait()
            pltpu.sync_copy(rows_v, g_h.at[pl.ds((base + s_local) * K, K)])
            return 0

        jax.lax.fori_loop(0, rows_per, row_body, 0)

    qb = jnp.broadcast_to(new_xyz.reshape(B * S, 1, 3), (B * S, 16, 3))
    g = bq(qb[..., 0].reshape(-1), qb[..., 1].reshape(-1),
           qb[..., 2].reshape(-1), xyz[..., 0].reshape(B * N),
           xyz[..., 1].reshape(B * N), xyz[..., 2].reshape(B * N),
           table.reshape(B * N, D))
    return g.reshape(B, S, K, D)


def _sa(xyz, feats, npoint, radius, K, params):
    B, N, _ = xyz.shape
    F = feats.shape[-1]
    D = -(-(3 + F) // 128) * 128
    inds, nxyzT = _fps(xyz, npoint)
    new_xyz = jnp.transpose(nxyzT, (0, 2, 1))
    table = jnp.concatenate(
        [xyz, feats, jnp.zeros((B, N, D - 3 - F), jnp.float32)], axis=-1)
    g = _ball_group(radius, K, xyz, new_xyz, table)
    q = jnp.concatenate(
        [new_xyz, jnp.zeros((B, npoint, D - 3), jnp.float32)], axis=-1)
    f = _mlp_max(g, q, params, D, radius)
    return new_xyz, f, inds, nxyzT


def kernel(pointcloud, params):
    xyz = pointcloud[:, :, 0:3]
    feats = pointcloud[:, :, 4:]
    x1, f1, i1, x1t = _sa(xyz, feats, 1024, 0.2, 64, params['sa1'])
    x2, f2, i2, x2t = _sa(x1, f1, 512, 0.4, 32, params['sa2'])
    x3, f3, i3, x3t = _sa(x2, f2, 256, 0.8, 16, params['sa3'])
    x4, f4, i4, x4t = _sa(x3, f3, 256, 1.2, 16, params['sa4'])
    f3p = _fp(x3, f3, x4t, f4, params['fp1'])
    f2p = _fp(x2, f2, x3t, f3p, params['fp2'])
    return x2, f2p, i2


# speedup vs baseline: 1.3375x; 1.3375x over previous
import functools

import jax
import jax.numpy as jnp
from jax.experimental import pallas as pl
from jax.experimental.pallas import tpu as pltpu
from jax.experimental.pallas import tpu_sc as plsc


def _fps_body(npoint, Nr, B, xyz_ref, inds_ref, nxyz_ref, dist_ref):
    r_io = jax.lax.broadcasted_iota(jnp.int32, (Nr, 128), 0)
    for b in range(B):
        dist_ref[b] = jnp.full((Nr, 128), 1e10, jnp.float32)
    l_io = jax.lax.broadcasted_iota(jnp.int32, (1, 128), 1)
    BIGI = jnp.int32(Nr * 128)

    def step(t, fars):
        tr = t // 128
        tc = t - tr * 128
        hitc = l_io == tc
        new_fars = []
        for b in range(B):
            far = fars[b]
            r = far // 128
            c = far - r * 128
            csel = l_io == c
            cx = jnp.sum(jnp.where(csel, xyz_ref[b, 0, pl.ds(r, 1), :], 0.0))
            cy = jnp.sum(jnp.where(csel, xyz_ref[b, 1, pl.ds(r, 1), :], 0.0))
            cz = jnp.sum(jnp.where(csel, xyz_ref[b, 2, pl.ds(r, 1), :], 0.0))
            inds_ref[b, pl.ds(tr, 1), :] = jnp.where(
                hitc, far, inds_ref[b, pl.ds(tr, 1), :])
            nxyz_ref[b, 0, pl.ds(tr, 1), :] = jnp.where(
                hitc, cx, nxyz_ref[b, 0, pl.ds(tr, 1), :])
            nxyz_ref[b, 1, pl.ds(tr, 1), :] = jnp.where(
                hitc, cy, nxyz_ref[b, 1, pl.ds(tr, 1), :])
            nxyz_ref[b, 2, pl.ds(tr, 1), :] = jnp.where(
                hitc, cz, nxyz_ref[b, 2, pl.ds(tr, 1), :])
            dx = xyz_ref[b, 0] - cx
            dy = xyz_ref[b, 1] - cy
            dz = xyz_ref[b, 2] - cz
            d = dx * dx + dy * dy + dz * dz
            nd = jnp.minimum(dist_ref[b], d)
            dist_ref[b] = nd
            rowmax = jnp.max(nd, axis=0, keepdims=True)
            rowarg = jnp.min(jnp.where(nd == rowmax, r_io, jnp.int32(Nr)),
                             axis=0, keepdims=True)
            m = jnp.max(rowmax)
            cand = jnp.where(rowmax == m, rowarg * 128 + l_io, BIGI)
            new_fars.append(jnp.min(cand))
        return tuple(new_fars)

    jax.lax.fori_loop(0, npoint, step, (jnp.int32(0),) * B)


def _fps(xyz, npoint):
    B, N, _ = xyz.shape
    Nr = max(8, -(-N // 128))
    Np = Nr * 128
    rows = npoint // 128
    if Np > N:
        pad = jnp.broadcast_to(xyz[:, 0:1, :], (B, Np - N, 3))
        xyz = jnp.concatenate([xyz, pad], axis=1)
    xyz_p = jnp.transpose(xyz, (0, 2, 1)).reshape(B, 3, Nr, 128)
    inds, nxyz = pl.pallas_call(
        functools.partial(_fps_body, npoint, Nr, B),
        grid=(1,),
        in_specs=[pl.BlockSpec((B, 3, Nr, 128), lambda i: (0, 0, 0, 0))],
        out_specs=[pl.BlockSpec((B, rows, 128), lambda i: (0, 0, 0)),
                   pl.BlockSpec((B, 3, rows, 128), lambda i: (0, 0, 0, 0))],
        out_shape=[jax.ShapeDtypeStruct((B, rows, 128), jnp.int32),
                   jax.ShapeDtypeStruct((B, 3, rows, 128), jnp.float32)],
        scratch_shapes=[pltpu.VMEM((B, Nr, 128), jnp.float32)],
    )(xyz_p)
    return inds.reshape(B, npoint), nxyz.reshape(B, 3, npoint)


def _mlp_body(K, inv_r, g_ref, q_ref, w1_ref, b1_ref, w2_ref, b2_ref,
              w3_ref, b3_ref, out_ref):
    SB = g_ref.shape[1]
    D = g_ref.shape[3]
    g = g_ref[0]
    q = q_ref[0][:, None, :]
    col = jax.lax.broadcasted_iota(jnp.int32, (SB, K, D), 2)
    h0 = jnp.where(col < 3, (g - q) * inv_r, g)
    xx = h0.reshape(SB * K, D)
    h = jnp.maximum(jnp.dot(xx, w1_ref[...],
                            preferred_element_type=jnp.float32) + b1_ref[...], 0.0)
    h = jnp.maximum(jnp.dot(h, w2_ref[...],
                            preferred_element_type=jnp.float32) + b2_ref[...], 0.0)
    h = jnp.maximum(jnp.dot(h, w3_ref[...],
                            preferred_element_type=jnp.float32) + b3_ref[...], 0.0)
    C3 = h.shape[-1]
    out_ref[0] = jnp.max(h.reshape(SB, K, C3), axis=1)


def _mlp_max(g, q, params, D, radius):
    B, S, K, _ = g.shape
    SB = min(S, 128)
    (w1, b1), (w2, b2), (w3, b3) = params
    cin = w1.shape[0]
    w1p = jnp.concatenate([w1, jnp.zeros((D - cin, w1.shape[1]), jnp.float32)], axis=0)
    C3 = w3.shape[1]
    out = pl.pallas_call(
        functools.partial(_mlp_body, K, 1.0 / radius),
        grid=(B, S // SB),
        in_specs=[
            pl.BlockSpec((1, SB, K, D), lambda b, s: (b, s, 0, 0)),
            pl.BlockSpec((1, SB, D), lambda b, s: (b, s, 0)),
            pl.BlockSpec(w1p.shape, lambda b, s: (0, 0)),
            pl.BlockSpec((1, b1.shape[0]), lambda b, s: (0, 0)),
            pl.BlockSpec(w2.shape, lambda b, s: (0, 0)),
            pl.BlockSpec((1, b2.shape[0]), lambda b, s: (0, 0)),
            pl.BlockSpec(w3.shape, lambda b, s: (0, 0)),
            pl.BlockSpec((1, b3.shape[0]), lambda b, s: (0, 0)),
        ],
        out_specs=pl.BlockSpec((1, SB, C3), lambda b, s: (b, s, 0)),
        out_shape=jax.ShapeDtypeStruct((B, S, C3), jnp.float32),
    )(g, q, w1p, b1[None, :], w2, b2[None, :], w3, b3[None, :])
    return out


def _fp_body(S2, f1_ref, x1_ref, x2t_ref, f2_ref, w1_ref, b1_ref,
             w2_ref, b2_ref, out_ref):
    x1 = x1_ref[0]
    x2 = x2t_ref[0]
    S1 = x1.shape[0]
    d = None
    for c in range(3):
        t = x1[:, c:c + 1] - x2[c:c + 1, :]
        t = t * t
        d = t if d is None else d + t
    colj = jax.lax.broadcasted_iota(jnp.int32, (S1, S2), 1)
    f2 = f2_ref[0]
    dd = d
    invs = []
    rows = []
    for _ in range(3):
        mn = jnp.min(dd, axis=1, keepdims=True)
        im = jnp.min(jnp.where(dd == mn, colj, S2), axis=1, keepdims=True)
        dd = jnp.where(colj == im, 1e30, dd)
        invs.append(1.0 / jnp.maximum(mn, 1e-10))
        oh = (colj == im).astype(jnp.float32)
        rows.append(jnp.dot(oh, f2, preferred_element_type=jnp.float32))
    wsum = (invs[0] + invs[1]) + invs[2]
    interp = ((rows[0] * (invs[0] / wsum) + rows[1] * (invs[1] / wsum))
              + rows[2] * (invs[2] / wsum))
    h = jnp.concatenate([interp, f1_ref[0]], axis=-1)
    h = jnp.maximum(jnp.dot(h, w1_ref[...],
                            preferred_element_type=jnp.float32) + b1_ref[...], 0.0)
    h = jnp.maximum(jnp.dot(h, w2_ref[...],
                            preferred_element_type=jnp.float32) + b2_ref[...], 0.0)
    out_ref[0] = h


def _fp(x1, f1, x2t, f2, params):
    B, S1, C1 = f1.shape
    S2 = f2.shape[1]
    (w1, b1), (w2, b2) = params
    Cout = w2.shape[1]
    out = pl.pallas_call(
        functools.partial(_fp_body, S2),
        grid=(B,),
        in_specs=[
            pl.BlockSpec((1, S1, C1), lambda b: (b, 0, 0)),
            pl.BlockSpec((1, S1, 3), lambda b: (b, 0, 0)),
            pl.BlockSpec((1, 3, S2), lambda b: (b, 0, 0)),
            pl.BlockSpec((1, S2, f2.shape[2]), lambda b: (b, 0, 0)),
            pl.BlockSpec(w1.shape, lambda b: (0, 0)),
            pl.BlockSpec((1, b1.shape[0]), lambda b: (0, 0)),
            pl.BlockSpec(w2.shape, lambda b: (0, 0)),
            pl.BlockSpec((1, b2.shape[0]), lambda b: (0, 0)),
        ],
        out_specs=pl.BlockSpec((1, S1, Cout), lambda b: (b, 0, 0)),
        out_shape=jax.ShapeDtypeStruct((B, S1, Cout), jnp.float32),
    )(f1, x1, x2t, f2, w1, b1[None, :], w2, b2[None, :])
    return out


def _ball_group(radius, K, xyz, new_xyz, table):
    B, S, _ = new_xyz.shape
    N = xyz.shape[1]
    D = table.shape[-1]
    NW = 32
    rows_per = (B * S) // NW
    nc = N // 16
    r2 = radius * radius
    mesh = plsc.VectorSubcoreMesh(core_axis_name="c", subcore_axis_name="s")

    @functools.partial(
        pl.kernel, mesh=mesh,
        compiler_params=pltpu.CompilerParams(needs_layout_passes=False),
        out_type=jax.ShapeDtypeStruct((B * S * K, D), jnp.float32),
        scratch_types=[
            pltpu.VMEM((N,), jnp.float32),
            pltpu.VMEM((N,), jnp.float32),
            pltpu.VMEM((N,), jnp.float32),
            pltpu.VMEM((rows_per * 16,), jnp.float32),
            pltpu.VMEM((rows_per * 16,), jnp.float32),
            pltpu.VMEM((rows_per * 16,), jnp.float32),
            pltpu.VMEM((K,), jnp.int32),
            pltpu.VMEM((K,), jnp.int32),
            pltpu.VMEM((K, D), jnp.float32),
            pltpu.VMEM((16,), jnp.int32),
            pltpu.SemaphoreType.DMA,
        ],
    )
    def bq(qx_h, qy_h, qz_h, px_h, py_h, pz_h, tab_h, g_h,
           xv, yv, zv, qxv, qyv, qzv, grp_v, gidx_v, rows_v, dbg_v, sem):
        wid = jax.lax.axis_index("s") * 2 + jax.lax.axis_index("c")
        base = wid * rows_per
        b = base // S
        pltpu.sync_copy(px_h.at[pl.ds(b * N, N)], xv)
        pltpu.sync_copy(py_h.at[pl.ds(b * N, N)], yv)
        pltpu.sync_copy(pz_h.at[pl.ds(b * N, N)], zv)
        pltpu.sync_copy(qx_h.at[pl.ds(base * 16, rows_per * 16)], qxv)
        pltpu.sync_copy(qy_h.at[pl.ds(base * 16, rows_per * 16)], qyv)
        pltpu.sync_copy(qz_h.at[pl.ds(base * 16, rows_per * 16)], qzv)
        iot = jax.lax.iota(jnp.int32, 16)
        zeros16 = jnp.zeros((16,), jnp.int32)
        bN = b * N

        def row_body(s_local, _):
            qoff = s_local * 16
            qxs = qxv[pl.ds(qoff, 16)]
            qys = qyv[pl.ds(qoff, 16)]
            qzs = qzv[pl.ds(qoff, 16)]

            def chunk(i, cnt_vec):
                off = i * 16
                dx = xv[pl.ds(off, 16)] - qxs
                dy = yv[pl.ds(off, 16)] - qys
                dz = zv[pl.ds(off, 16)] - qzs
                dd = dx * dx + dy * dy + dz * dz
                m = dd <= r2
                pc = plsc.all_reduce_population_count(m)
                c = plsc.cumsum(jnp.where(m, jnp.int32(1), jnp.int32(0)))
                pos = cnt_vec + c - 1
                keep = m & (pos < K)
                pos_st = jnp.where(keep, pos, jnp.int32(0))
                plsc.store_scatter(grp_v, [pos_st], iot + off, mask=keep)
                return jnp.minimum(cnt_vec + pc, jnp.int32(K))

            cnt = jax.lax.fori_loop(0, nc, chunk, zeros16)
            v0vec = grp_v[pl.ds(0, 16)]
            v0 = plsc.cummax(jnp.where(iot == 0, v0vec, jnp.int32(-1)))
            for kb in range(K // 16):
                sl = pl.ds(kb * 16, 16)
                sel = (iot + kb * 16) >= cnt
                gidx_v[sl] = jnp.where(sel, v0, grp_v[sl]) + bN
            pltpu.async_copy(tab_h.at[gidx_v], rows_v, sem).wait()
            pltpu.sync_copy(rows_v, g_h.at[pl.ds((base + s_local) * K, K)])
            return 0

        jax.lax.fori_loop(0, rows_per, row_body, 0)

    qb = jnp.broadcast_to(new_xyz.reshape(B * S, 1, 3), (B * S, 16, 3))
    g = bq(qb[..., 0].reshape(-1), qb[..., 1].reshape(-1),
           qb[..., 2].reshape(-1), xyz[..., 0].reshape(B * N),
           xyz[..., 1].reshape(B * N), xyz[..., 2].reshape(B * N),
           table.reshape(B * N, D))
    return g.reshape(B, S, K, D)


def _sa(xyz, feats, npoint, radius, K, params):
    B, N, _ = xyz.shape
    F = feats.shape[-1]
    D = -(-(3 + F) // 128) * 128
    inds, nxyzT = _fps(xyz, npoint)
    new_xyz = jnp.transpose(nxyzT, (0, 2, 1))
    table = jnp.concatenate(
        [xyz, feats, jnp.zeros((B, N, D - 3 - F), jnp.float32)], axis=-1)
    g = _ball_group(radius, K, xyz, new_xyz, table)
    q = jnp.concatenate(
        [new_xyz, jnp.zeros((B, npoint, D - 3), jnp.float32)], axis=-1)
    f = _mlp_max(g, q, params, D, radius)
    return new_xyz, f, inds, nxyzT


def kernel(pointcloud, params):
    xyz = pointcloud[:, :, 0:3]
    feats = pointcloud[:, :, 4:]
    x1, f1, i1, x1t = _sa(xyz, feats, 1024, 0.2, 64, params['sa1'])
    x2, f2, i2, x2t = _sa(x1, f1, 512, 0.4, 32, params['sa2'])
    x3, f3, i3, x3t = _sa(x2, f2, 256, 0.8, 16, params['sa3'])
    x4, f4, i4, x4t = _sa(x3, f3, 256, 1.2, 16, params['sa4'])
    f3p = _fp(x3, f3, x4t, f4, params['fp1'])
    f2p = _fp(x2, f2, x3t, f3p, params['fp2'])
    return x2, f2p, i2


# speedup vs baseline: 1.3766x; 1.0292x over previous
import functools

import jax
import jax.numpy as jnp
from jax.experimental import pallas as pl
from jax.experimental.pallas import tpu as pltpu
from jax.experimental.pallas import tpu_sc as plsc


def _fps_body(npoint, Nr, B, xyz_ref, inds_ref, nxyz_ref, dist_ref):
    r_io = jax.lax.broadcasted_iota(jnp.int32, (Nr, 128), 0)
    for b in range(B):
        dist_ref[b] = jnp.full((Nr, 128), 1e10, jnp.float32)
    l_io = jax.lax.broadcasted_iota(jnp.int32, (1, 128), 1)
    BIGI = jnp.int32(Nr * 128)

    def step(t, fars):
        tr = t // 128
        tc = t - tr * 128
        hitc = l_io == tc
        new_fars = []
        for b in range(B):
            far = fars[b]
            r = far // 128
            c = far - r * 128
            csel = l_io == c
            cx = jnp.sum(jnp.where(csel, xyz_ref[b, 0, pl.ds(r, 1), :], 0.0))
            cy = jnp.sum(jnp.where(csel, xyz_ref[b, 1, pl.ds(r, 1), :], 0.0))
            cz = jnp.sum(jnp.where(csel, xyz_ref[b, 2, pl.ds(r, 1), :], 0.0))
            inds_ref[b, pl.ds(tr, 1), :] = jnp.where(
                hitc, far, inds_ref[b, pl.ds(tr, 1), :])
            nxyz_ref[b, 0, pl.ds(tr, 1), :] = jnp.where(
                hitc, cx, nxyz_ref[b, 0, pl.ds(tr, 1), :])
            nxyz_ref[b, 1, pl.ds(tr, 1), :] = jnp.where(
                hitc, cy, nxyz_ref[b, 1, pl.ds(tr, 1), :])
            nxyz_ref[b, 2, pl.ds(tr, 1), :] = jnp.where(
                hitc, cz, nxyz_ref[b, 2, pl.ds(tr, 1), :])
            dx = xyz_ref[b, 0] - cx
            dy = xyz_ref[b, 1] - cy
            dz = xyz_ref[b, 2] - cz
            d = dx * dx + dy * dy + dz * dz
            nd = jnp.minimum(dist_ref[b], d)
            dist_ref[b] = nd
            rowmax = jnp.max(nd, axis=0, keepdims=True)
            rowarg = jnp.min(jnp.where(nd == rowmax, r_io, jnp.int32(Nr)),
                             axis=0, keepdims=True)
            m = jnp.max(rowmax)
            cand = jnp.where(rowmax == m, rowarg * 128 + l_io, BIGI)
            new_fars.append(jnp.min(cand))
        return tuple(new_fars)

    jax.lax.fori_loop(0, npoint, step, (jnp.int32(0),) * B)


def _fps(xyz, npoint):
    B, N, _ = xyz.shape
    Nr = max(8, -(-N // 128))
    Np = Nr * 128
    rows = npoint // 128
    if Np > N:
        pad = jnp.broadcast_to(xyz[:, 0:1, :], (B, Np - N, 3))
        xyz = jnp.concatenate([xyz, pad], axis=1)
    xyz_p = jnp.transpose(xyz, (0, 2, 1)).reshape(B, 3, Nr, 128)
    inds, nxyz = pl.pallas_call(
        functools.partial(_fps_body, npoint, Nr, B),
        grid=(1,),
        in_specs=[pl.BlockSpec((B, 3, Nr, 128), lambda i: (0, 0, 0, 0))],
        out_specs=[pl.BlockSpec((B, rows, 128), lambda i: (0, 0, 0)),
                   pl.BlockSpec((B, 3, rows, 128), lambda i: (0, 0, 0, 0))],
        out_shape=[jax.ShapeDtypeStruct((B, rows, 128), jnp.int32),
                   jax.ShapeDtypeStruct((B, 3, rows, 128), jnp.float32)],
        scratch_shapes=[pltpu.VMEM((B, Nr, 128), jnp.float32)],
    )(xyz_p)
    return inds.reshape(B, npoint), nxyz.reshape(B, 3, npoint)


def _mlp_body(K, inv_r, g_ref, q_ref, w1_ref, b1_ref, w2_ref, b2_ref,
              w3_ref, b3_ref, out_ref):
    SB = g_ref.shape[1]
    D = g_ref.shape[3]
    g = g_ref[0]
    q = q_ref[0][:, None, :]
    col = jax.lax.broadcasted_iota(jnp.int32, (SB, K, D), 2)
    h0 = jnp.where(col < 3, (g - q) * inv_r, g)
    xx = h0.reshape(SB * K, D)
    h = jnp.maximum(jnp.dot(xx, w1_ref[...],
                            preferred_element_type=jnp.float32) + b1_ref[...], 0.0)
    h = jnp.maximum(jnp.dot(h, w2_ref[...],
                            preferred_element_type=jnp.float32) + b2_ref[...], 0.0)
    h = jnp.maximum(jnp.dot(h, w3_ref[...],
                            preferred_element_type=jnp.float32) + b3_ref[...], 0.0)
    C3 = h.shape[-1]
    out_ref[0] = jnp.max(h.reshape(SB, K, C3), axis=1)


def _mlp_max(g, q, params, D, radius):
    B, S, K, _ = g.shape
    SB = min(S, 128)
    (w1, b1), (w2, b2), (w3, b3) = params
    cin = w1.shape[0]
    w1p = jnp.concatenate([w1, jnp.zeros((D - cin, w1.shape[1]), jnp.float32)], axis=0)
    C3 = w3.shape[1]
    out = pl.pallas_call(
        functools.partial(_mlp_body, K, 1.0 / radius),
        grid=(B, S // SB),
        in_specs=[
            pl.BlockSpec((1, SB, K, D), lambda b, s: (b, s, 0, 0)),
            pl.BlockSpec((1, SB, D), lambda b, s: (b, s, 0)),
            pl.BlockSpec(w1p.shape, lambda b, s: (0, 0)),
            pl.BlockSpec((1, b1.shape[0]), lambda b, s: (0, 0)),
            pl.BlockSpec(w2.shape, lambda b, s: (0, 0)),
            pl.BlockSpec((1, b2.shape[0]), lambda b, s: (0, 0)),
            pl.BlockSpec(w3.shape, lambda b, s: (0, 0)),
            pl.BlockSpec((1, b3.shape[0]), lambda b, s: (0, 0)),
        ],
        out_specs=pl.BlockSpec((1, SB, C3), lambda b, s: (b, s, 0)),
        out_shape=jax.ShapeDtypeStruct((B, S, C3), jnp.float32),
    )(g, q, w1p, b1[None, :], w2, b2[None, :], w3, b3[None, :])
    return out


def _fp_body(S2, f1_ref, x1_ref, x2t_ref, f2_ref, w1_ref, b1_ref,
             w2_ref, b2_ref, out_ref):
    x1 = x1_ref[0]
    x2 = x2t_ref[0]
    S1 = x1.shape[0]
    d = None
    for c in range(3):
        t = x1[:, c:c + 1] - x2[c:c + 1, :]
        t = t * t
        d = t if d is None else d + t
    colj = jax.lax.broadcasted_iota(jnp.int32, (S1, S2), 1)
    f2 = f2_ref[0]
    dd = d
    invs = []
    rows = []
    for _ in range(3):
        mn = jnp.min(dd, axis=1, keepdims=True)
        im = jnp.min(jnp.where(dd == mn, colj, S2), axis=1, keepdims=True)
        dd = jnp.where(colj == im, 1e30, dd)
        invs.append(1.0 / jnp.maximum(mn, 1e-10))
        oh = (colj == im).astype(jnp.float32)
        rows.append(jnp.dot(oh, f2, preferred_element_type=jnp.float32))
    wsum = (invs[0] + invs[1]) + invs[2]
    interp = ((rows[0] * (invs[0] / wsum) + rows[1] * (invs[1] / wsum))
              + rows[2] * (invs[2] / wsum))
    h = jnp.concatenate([interp, f1_ref[0]], axis=-1)
    h = jnp.maximum(jnp.dot(h, w1_ref[...],
                            preferred_element_type=jnp.float32) + b1_ref[...], 0.0)
    h = jnp.maximum(jnp.dot(h, w2_ref[...],
                            preferred_element_type=jnp.float32) + b2_ref[...], 0.0)
    out_ref[0] = h


def _fp(x1, f1, x2t, f2, params):
    B, S1, C1 = f1.shape
    S2 = f2.shape[1]
    (w1, b1), (w2, b2) = params
    Cout = w2.shape[1]
    out = pl.pallas_call(
        functools.partial(_fp_body, S2),
        grid=(B,),
        in_specs=[
            pl.BlockSpec((1, S1, C1), lambda b: (b, 0, 0)),
            pl.BlockSpec((1, S1, 3), lambda b: (b, 0, 0)),
            pl.BlockSpec((1, 3, S2), lambda b: (b, 0, 0)),
            pl.BlockSpec((1, S2, f2.shape[2]), lambda b: (b, 0, 0)),
            pl.BlockSpec(w1.shape, lambda b: (0, 0)),
            pl.BlockSpec((1, b1.shape[0]), lambda b: (0, 0)),
            pl.BlockSpec(w2.shape, lambda b: (0, 0)),
            pl.BlockSpec((1, b2.shape[0]), lambda b: (0, 0)),
        ],
        out_specs=pl.BlockSpec((1, S1, Cout), lambda b: (b, 0, 0)),
        out_shape=jax.ShapeDtypeStruct((B, S1, Cout), jnp.float32),
    )(f1, x1, x2t, f2, w1, b1[None, :], w2, b2[None, :])
    return out


def _ball_group(radius, K, xyz, new_xyz, table):
    B, S, _ = new_xyz.shape
    N = xyz.shape[1]
    D = table.shape[-1]
    NW = 32
    rows_per = (B * S) // NW
    nc = N // 16
    r2 = radius * radius
    mesh = plsc.VectorSubcoreMesh(core_axis_name="c", subcore_axis_name="s")

    @functools.partial(
        pl.kernel, mesh=mesh,
        compiler_params=pltpu.CompilerParams(needs_layout_passes=False),
        out_type=jax.ShapeDtypeStruct((B * S * K, D), jnp.float32),
        scratch_types=[
            pltpu.VMEM((N,), jnp.float32),
            pltpu.VMEM((N,), jnp.float32),
            pltpu.VMEM((N,), jnp.float32),
            pltpu.VMEM((rows_per * 16,), jnp.float32),
            pltpu.VMEM((rows_per * 16,), jnp.float32),
            pltpu.VMEM((rows_per * 16,), jnp.float32),
            pltpu.VMEM((K,), jnp.int32),
            pltpu.VMEM((K,), jnp.int32),
            pltpu.VMEM((K, D), jnp.float32),
            pltpu.VMEM((16,), jnp.int32),
            pltpu.SemaphoreType.DMA,
        ],
    )
    def bq(qx_h, qy_h, qz_h, px_h, py_h, pz_h, tab_h, g_h,
           xv, yv, zv, qxv, qyv, qzv, grp_v, gidx_v, rows_v, dbg_v, sem):
        wid = jax.lax.axis_index("s") * 2 + jax.lax.axis_index("c")
        base = wid * rows_per
        b = base // S
        pltpu.sync_copy(px_h.at[pl.ds(b * N, N)], xv)
        pltpu.sync_copy(py_h.at[pl.ds(b * N, N)], yv)
        pltpu.sync_copy(pz_h.at[pl.ds(b * N, N)], zv)
        pltpu.sync_copy(qx_h.at[pl.ds(base * 16, rows_per * 16)], qxv)
        pltpu.sync_copy(qy_h.at[pl.ds(base * 16, rows_per * 16)], qyv)
        pltpu.sync_copy(qz_h.at[pl.ds(base * 16, rows_per * 16)], qzv)
        iot = jax.lax.iota(jnp.int32, 16)
        zeros16 = jnp.zeros((16,), jnp.int32)
        bN = b * N

        def row_body(s_local, _):
            qoff = s_local * 16
            qxs = qxv[pl.ds(qoff, 16)]
            qys = qyv[pl.ds(qoff, 16)]
            qzs = qzv[pl.ds(qoff, 16)]

            def chunk(i, cnt_vec):
                for u in range(2):
                    off = i * 32 + u * 16
                    dx = xv[pl.ds(off, 16)] - qxs
                    dy = yv[pl.ds(off, 16)] - qys
                    dz = zv[pl.ds(off, 16)] - qzs
                    dd = dx * dx + dy * dy + dz * dz
                    m = dd <= r2
                    c = plsc.cumsum(jnp.where(m, jnp.int32(1), jnp.int32(0)))
                    pos = cnt_vec + c - 1
                    keep = m & (pos < K)
                    pos_st = jnp.where(keep, pos, jnp.int32(0))
                    plsc.store_scatter(grp_v, [pos_st], iot + off, mask=keep)
                    cnt_vec = cnt_vec + plsc.all_reduce_population_count(keep)
                return cnt_vec

            cnt = jax.lax.fori_loop(0, nc // 2, chunk, zeros16)
            v0vec = grp_v[pl.ds(0, 16)]
            v0 = plsc.cummax(jnp.where(iot == 0, v0vec, jnp.int32(-1)))
            for kb in range(K // 16):
                sl = pl.ds(kb * 16, 16)
                sel = (iot + kb * 16) >= cnt
                gidx_v[sl] = jnp.where(sel, v0, grp_v[sl]) + bN
            pltpu.async_copy(tab_h.at[gidx_v], rows_v, sem).wait()
            pltpu.sync_copy(rows_v, g_h.at[pl.ds((base + s_local) * K, K)])
            return 0

        jax.lax.fori_loop(0, rows_per, row_body, 0)

    qb = jnp.broadcast_to(new_xyz.reshape(B * S, 1, 3), (B * S, 16, 3))
    g = bq(qb[..., 0].reshape(-1), qb[..., 1].reshape(-1),
           qb[..., 2].reshape(-1), xyz[..., 0].reshape(B * N),
           xyz[..., 1].reshape(B * N), xyz[..., 2].reshape(B * N),
           table.reshape(B * N, D))
    return g.reshape(B, S, K, D)


def _sa(xyz, feats, npoint, radius, K, params):
    B, N, _ = xyz.shape
    F = feats.shape[-1]
    D = -(-(3 + F) // 128) * 128
    inds, nxyzT = _fps(xyz, npoint)
    new_xyz = jnp.transpose(nxyzT, (0, 2, 1))
    table = jnp.concatenate(
        [xyz, feats, jnp.zeros((B, N, D - 3 - F), jnp.float32)], axis=-1)
    g = _ball_group(radius, K, xyz, new_xyz, table)
    q = jnp.concatenate(
        [new_xyz, jnp.zeros((B, npoint, D - 3), jnp.float32)], axis=-1)
    f = _mlp_max(g, q, params, D, radius)
    return new_xyz, f, inds, nxyzT


def kernel(pointcloud, params):
    xyz = pointcloud[:, :, 0:3]
    feats = pointcloud[:, :, 4:]
    x1, f1, i1, x1t = _sa(xyz, feats, 1024, 0.2, 64, params['sa1'])
    x2, f2, i2, x2t = _sa(x1, f1, 512, 0.4, 32, params['sa2'])
    x3, f3, i3, x3t = _sa(x2, f2, 256, 0.8, 16, params['sa3'])
    x4, f4, i4, x4t = _sa(x3, f3, 256, 1.2, 16, params['sa4'])
    f3p = _fp(x3, f3, x4t, f4, params['fp1'])
    f2p = _fp(x2, f2, x3t, f3p, params['fp2'])
    return x2, f2p, i2
